# Initial kernel scaffold; baseline (speedup 1.0000x reference)
import jax
import jax.numpy as jnp
from jax.experimental import pallas as pl


def kernel(inputs, embedding):
    raise NotImplementedError("write your pallas kernel here")


# speedup vs baseline: 1.2962x; 1.2962x over previous
import functools

import jax
import jax.numpy as jnp
from jax import lax
from jax.experimental import pallas as pl
from jax.experimental.pallas import tpu as pltpu
from jax.experimental.pallas import tpu_sc as plsc

B, L = 4096, 20
VOCAB = 1000
D = 1000
DPAD = 1024
NC, NS = 2, 16
NW = NC * NS
B_PER_W = B // NW


def _sc_gather(idx, table_pad):
    mesh = plsc.VectorSubcoreMesh(core_axis_name="c", subcore_axis_name="s")

    @functools.partial(
        pl.kernel,
        out_type=jax.ShapeDtypeStruct((B, L, D), jnp.float32),
        mesh=mesh,
        scratch_types=[
            pltpu.VMEM((B_PER_W, L), jnp.int32),
            pltpu.VMEM((L, D), jnp.float32),
            pltpu.SemaphoreType.DMA,
        ],
        compiler_params=pltpu.CompilerParams(use_tc_tiling_on_sc=False),
    )
    def k(idx_hbm, table_hbm, out_hbm, idx_v, rows_v, gsem):
        wid = lax.axis_index("s") * NC + lax.axis_index("c")
        b0 = wid * B_PER_W
        pltpu.sync_copy(idx_hbm.at[pl.ds(b0, B_PER_W)], idx_v)

        def body(i, carry):
            pltpu.async_copy(table_hbm.at[idx_v.at[i]], rows_v, gsem).wait()
            pltpu.sync_copy(rows_v, out_hbm.at[b0 + i])
            return carry

        lax.fori_loop(0, B_PER_W, body, 0)

    return k(idx, table_pad)


def kernel(inputs, embedding):
    idx = inputs.astype(jnp.int32)
    return _sc_gather(idx, embedding)


# speedup vs baseline: 1.9180x; 1.4796x over previous
import functools

import jax
import jax.numpy as jnp
from jax import lax
from jax.experimental import pallas as pl
from jax.experimental.pallas import tpu as pltpu
from jax.experimental.pallas import tpu_sc as plsc

B, L = 4096, 20
VOCAB = 1000
D = 1000
DPAD = 1024
DMAIN = 896
DTAIL = D - DMAIN
NC, NS = 2, 16
NW = NC * NS
B_PER_W = B // NW


def _sc_gather(idx, table_pad):
    mesh = plsc.VectorSubcoreMesh(core_axis_name="c", subcore_axis_name="s")

    @functools.partial(
        pl.kernel,
        out_type=jax.ShapeDtypeStruct((B, L, D), jnp.float32),
        mesh=mesh,
        scratch_types=[
            pltpu.VMEM((B_PER_W, L), jnp.int32),
            pltpu.VMEM((2, L, DPAD), jnp.float32),
            pltpu.VMEM((2, L, DTAIL), jnp.float32),
            pltpu.SemaphoreType.DMA((2,)),
        ],
    )
    def k(idx_hbm, table_hbm, out_hbm, idx_v, bufs, tails, gsem):
        wid = lax.axis_index("s") * NC + lax.axis_index("c")
        b0 = wid * B_PER_W
        pltpu.sync_copy(idx_hbm.at[pl.ds(b0, B_PER_W)], idx_v)

        def gather(i, p):
            pltpu.make_async_copy(
                table_hbm.at[idx_v.at[i]], bufs.at[p], gsem.at[p]
            ).start()

        gather(0, 0)

        def body(i, carry):
            p = lax.rem(i, 2)
            pltpu.make_async_copy(
                table_hbm.at[idx_v.at[i]], bufs.at[p], gsem.at[p]
            ).wait()

            @pl.when(i + 1 < B_PER_W)
            def _():
                gather(i + 1, 1 - p)

            for r in range(L):
                for c in (0, 16, 32, 48, 64, 80, 88):
                    tails[p, r, pl.ds(c, 16)] = bufs[p, r, pl.ds(DMAIN + c, 16)]

            pltpu.sync_copy(
                bufs.at[p, :, pl.ds(0, DMAIN)],
                out_hbm.at[b0 + i, :, pl.ds(0, DMAIN)],
            )
            pltpu.sync_copy(
                tails.at[p], out_hbm.at[b0 + i, :, pl.ds(DMAIN, DTAIL)]
            )
            return carry

        lax.fori_loop(0, B_PER_W, body, 0)

    return k(idx, table_pad)


def kernel(inputs, embedding):
    idx = inputs.astype(jnp.int32)
    table_pad = jnp.pad(embedding, ((0, 0), (0, DPAD - D)))
    return _sc_gather(idx, table_pad)
